# Initial kernel scaffold; baseline (speedup 1.0000x reference)
import jax
import jax.numpy as jnp
from jax.experimental import pallas as pl


def kernel(node_embedding, edge_embedding, edge_index, W1, b1, bn_gamma, bn_beta, W2, b2, W3, b3):
    raise NotImplementedError("write your pallas kernel here")


# speedup vs baseline: 2.1616x; 2.1616x over previous
import functools

import jax
import jax.numpy as jnp
from jax import lax
from jax.experimental import pallas as pl
from jax.experimental.pallas import tpu as pltpu
import jax.experimental.pallas.tpu_sc as plsc

N = 10000
E = 320000
D = 128
DE = 16
DIN = 2 * D + DE
H = 512
DO = 3 * D
EPS = 1e-5

NC = 2
NS = 16
NW = NC * NS
EPW = E // NW
CH = 80
NCHUNK = EPW // CH
BE = 1600
GRID_E = E // BE
BN = 1000
GRID_N = N // BN

@functools.cache
def _make_sc_gather():
    mesh = plsc.VectorSubcoreMesh(core_axis_name="c", subcore_axis_name="s")
    return functools.partial(
        pl.kernel,
        out_type=[jax.ShapeDtypeStruct((E, D), jnp.float32),
                  jax.ShapeDtypeStruct((E, D), jnp.float32)],
        mesh=mesh,
        scratch_types=[pltpu.VMEM((NCHUNK, CH), jnp.int32),
                       pltpu.VMEM((NCHUNK, CH), jnp.int32),
                       pltpu.VMEM((CH, D), jnp.float32),
                       pltpu.VMEM((CH, D), jnp.float32),
                       pltpu.SemaphoreType.DMA,
                       pltpu.SemaphoreType.DMA],
    )(_sc_gather_body)


def _sc_gather_body(x_hbm, is_hbm, id_hbm, xs_out, xd_out,
                    is_v, id_v, rs_v, rd_v, sem_s, sem_d):
    wid = lax.axis_index("s") * NC + lax.axis_index("c")
    pltpu.sync_copy(is_hbm.at[wid], is_v)
    pltpu.sync_copy(id_hbm.at[wid], id_v)
    base = wid * EPW

    def body(j, carry):
        off = base + j * CH
        pltpu.async_copy(x_hbm.at[is_v.at[j]], rs_v, sem_s).wait()
        pltpu.sync_copy(rs_v, xs_out.at[pl.ds(off, CH)])
        pltpu.async_copy(x_hbm.at[id_v.at[j]], rd_v, sem_d).wait()
        pltpu.sync_copy(rd_v, xd_out.at[pl.ds(off, CH)])
        return carry

    lax.fori_loop(0, NCHUNK, body, 0)


EPT = E // NS
NCHUNK2 = EPT // CH
HALF = 5120
NPR = 5248
RPT = NPR // NS


@functools.cache
def _make_sc_scatter():
    mesh = plsc.VectorSubcoreMesh(core_axis_name="c", subcore_axis_name="s")
    return functools.partial(
        pl.kernel,
        out_type=jax.ShapeDtypeStruct((NC, NPR, D), jnp.float32),
        mesh=mesh,
        scratch_types=[pltpu.VMEM((NCHUNK2, CH), jnp.int32),
                       pltpu.VMEM((NCHUNK2, CH), jnp.int32),
                       pltpu.VMEM((CH, D), jnp.float32),
                       pltpu.VMEM((CH, D), jnp.float32),
                       pltpu.VMEM_SHARED((NPR, D), jnp.float32)],
    )(_sc_scatter_body)


def _sc_scatter_body(se_hbm, de_hbm, is2, id2, z_hbm, out,
                     is_v, id_v, rs_v, rd_v, acc):
    cid = lax.axis_index("c")
    tid = lax.axis_index("s")
    stripe = pl.ds(tid * RPT, RPT)
    pltpu.sync_copy(z_hbm.at[stripe], acc.at[stripe])
    pltpu.sync_copy(is2.at[cid, tid], is_v)
    pltpu.sync_copy(id2.at[cid, tid], id_v)
    plsc.subcore_barrier()
    base = tid * EPT

    def body_s(j, carry):
        off = base + j * CH
        pltpu.sync_copy(se_hbm.at[pl.ds(off, CH)], rs_v)
        pltpu.sync_copy(rs_v, acc.at[is_v.at[j]], add=True)
        return carry

    lax.fori_loop(0, NCHUNK2, body_s, 0)

    def body_d(j, carry):
        off = base + j * CH
        pltpu.sync_copy(de_hbm.at[pl.ds(off, CH)], rd_v)
        pltpu.sync_copy(rd_v, acc.at[id_v.at[j]], add=True)
        return carry

    lax.fori_loop(0, NCHUNK2, body_d, 0)
    plsc.subcore_barrier()
    pltpu.sync_copy(acc.at[stripe], out.at[cid, stripe])


IR = E // D


def _tc_idx_body(i_ref, o_ref):
    i = i_ref[...]
    o_ref[0] = jnp.where(i < HALF, i, HALF)
    o_ref[1] = jnp.where(i >= HALF, i - HALF, HALF)


def _tc_idx(idx):
    return pl.pallas_call(
        _tc_idx_body,
        in_specs=[pl.BlockSpec((2, IR, D), lambda: (0, 0, 0))],
        out_specs=pl.BlockSpec((2, 2, IR, D), lambda: (0, 0, 0, 0)),
        out_shape=jax.ShapeDtypeStruct((2, 2, IR, D), jnp.int32),
    )(idx.reshape(2, IR, D))


def _tc_h_stats_body(xs_ref, e_ref, xd_ref, w1_ref, b1_ref,
                     h_ref, sums_ref, acc_ref):
    i = pl.program_id(0)
    h = (jnp.dot(xs_ref[...], w1_ref[0:D, :], preferred_element_type=jnp.float32)
         + jnp.dot(e_ref[...], w1_ref[D:D + DE, :], preferred_element_type=jnp.float32)
         + jnp.dot(xd_ref[...], w1_ref[D + DE:DIN, :], preferred_element_type=jnp.float32)
         + b1_ref[...])
    h_ref[...] = h

    @pl.when(i == 0)
    def _():
        acc_ref[...] = jnp.zeros_like(acc_ref)

    acc_ref[0:1, :] += jnp.sum(h, axis=0, keepdims=True)
    acc_ref[1:2, :] += jnp.sum(h * h, axis=0, keepdims=True)

    @pl.when(i == GRID_E - 1)
    def _():
        sums_ref[...] = acc_ref[...]


def _tc_h_stats(xs, e, xd, w1, b1):
    return pl.pallas_call(
        _tc_h_stats_body,
        grid=(GRID_E,),
        in_specs=[
            pl.BlockSpec((BE, D), lambda i: (i, 0)),
            pl.BlockSpec((BE, DE), lambda i: (i, 0)),
            pl.BlockSpec((BE, D), lambda i: (i, 0)),
            pl.BlockSpec((DIN, H), lambda i: (0, 0)),
            pl.BlockSpec((1, H), lambda i: (0, 0)),
        ],
        out_specs=[
            pl.BlockSpec((BE, H), lambda i: (i, 0)),
            pl.BlockSpec((8, H), lambda i: (0, 0)),
        ],
        out_shape=[jax.ShapeDtypeStruct((E, H), jnp.float32),
                   jax.ShapeDtypeStruct((8, H), jnp.float32)],
        scratch_shapes=[pltpu.VMEM((8, H), jnp.float32)],
    )(xs, e, xd, w1, b1)


def _tc_norm_body(h_ref, sums_ref, g_ref, bt_ref, w2_ref, b2_ref,
                  se_ref, ee_ref, de_ref):
    mean = sums_ref[0:1, :] * (1.0 / E)
    var = sums_ref[1:2, :] * (1.0 / E) - mean * mean
    alpha = g_ref[...] * lax.rsqrt(var + EPS)
    delta = bt_ref[...] - mean * alpha
    hn = jnp.maximum(h_ref[...] * alpha + delta, 0.0)
    o = jnp.dot(hn, w2_ref[...], preferred_element_type=jnp.float32) + b2_ref[...]
    se_ref[...] = o[:, 0:D]
    ee_ref[...] = o[:, D:2 * D]
    de_ref[...] = o[:, 2 * D:3 * D]


def _tc_norm(h, sums, g, bt, w2, b2):
    return pl.pallas_call(
        _tc_norm_body,
        grid=(GRID_E,),
        in_specs=[
            pl.BlockSpec((BE, H), lambda i: (i, 0)),
            pl.BlockSpec((8, H), lambda i: (0, 0)),
            pl.BlockSpec((1, H), lambda i: (0, 0)),
            pl.BlockSpec((1, H), lambda i: (0, 0)),
            pl.BlockSpec((H, DO), lambda i: (0, 0)),
            pl.BlockSpec((1, DO), lambda i: (0, 0)),
        ],
        out_specs=[
            pl.BlockSpec((BE, D), lambda i: (i, 0)),
            pl.BlockSpec((BE, D), lambda i: (i, 0)),
            pl.BlockSpec((BE, D), lambda i: (i, 0)),
        ],
        out_shape=[jax.ShapeDtypeStruct((E, D), jnp.float32),
                   jax.ShapeDtypeStruct((E, D), jnp.float32),
                   jax.ShapeDtypeStruct((E, D), jnp.float32)],
    )(h, sums, g, bt, w2, b2)


def _tc_final_body(p_ref, w3_ref, b3_ref, o_ref):
    o_ref[...] = (jnp.dot(p_ref[...], w3_ref[...],
                          preferred_element_type=jnp.float32)
                  + b3_ref[...])


def _tc_final(p, w3, b3):
    return pl.pallas_call(
        _tc_final_body,
        grid=(GRID_N,),
        in_specs=[
            pl.BlockSpec((BN, D), lambda i: (i, 0)),
            pl.BlockSpec((D, D), lambda i: (0, 0)),
            pl.BlockSpec((1, D), lambda i: (0, 0)),
        ],
        out_specs=pl.BlockSpec((BN, D), lambda i: (i, 0)),
        out_shape=jax.ShapeDtypeStruct((N, D), jnp.float32),
    )(p, w3, b3)


def kernel(node_embedding, edge_embedding, edge_index,
           W1, b1, bn_gamma, bn_beta, W2, b2, W3, b3):
    idx_s3 = edge_index[0].reshape(NW, NCHUNK, CH)
    idx_d3 = edge_index[1].reshape(NW, NCHUNK, CH)
    xs, xd = _make_sc_gather()(node_embedding, idx_s3, idx_d3)
    h, sums = _tc_h_stats(xs, edge_embedding, xd, W1, b1.reshape(1, H))
    src_e, edg_e, dst_e = _tc_norm(h, sums, bn_gamma.reshape(1, H),
                                   bn_beta.reshape(1, H), W2, b2.reshape(1, DO))
    idx2 = _tc_idx(edge_index)
    sh = (NC, NS, NCHUNK2, CH)
    is2 = idx2[:, 0].reshape(sh)
    id2 = idx2[:, 1].reshape(sh)
    zeros_n = jnp.zeros((NPR, D), jnp.float32)
    parts = _make_sc_scatter()(src_e, dst_e, is2, id2, zeros_n)
    p = jnp.concatenate([parts[0, :HALF], parts[1, :N - HALF]], axis=0)
    node_out = _tc_final(p, W3, b3.reshape(1, D))
    return node_out, edg_e


# speedup vs baseline: 2.6685x; 1.2345x over previous
import functools

import jax
import jax.numpy as jnp
from jax import lax
from jax.experimental import pallas as pl
from jax.experimental.pallas import tpu as pltpu
import jax.experimental.pallas.tpu_sc as plsc

N = 10000
E = 320000
D = 128
DE = 16
DIN = 2 * D + DE
H = 512
DO = 3 * D
EPS = 1e-5

NC = 2
NS = 16
NW = NC * NS
EPW = E // NW
CH = 80
NCHUNK = EPW // CH
BE = 1600
GRID_E = E // BE
BN = 1000
GRID_N = N // BN

@functools.cache
def _make_sc_gather():
    mesh = plsc.VectorSubcoreMesh(core_axis_name="c", subcore_axis_name="s")
    return functools.partial(
        pl.kernel,
        out_type=[jax.ShapeDtypeStruct((E, D), jnp.float32),
                  jax.ShapeDtypeStruct((E, D), jnp.float32)],
        mesh=mesh,
        scratch_types=[pltpu.VMEM((NCHUNK, CH), jnp.int32),
                       pltpu.VMEM((NCHUNK, CH), jnp.int32),
                       pltpu.VMEM((2, CH, D), jnp.float32),
                       pltpu.VMEM((2, CH, D), jnp.float32),
                       pltpu.SemaphoreType.DMA((2,)),
                       pltpu.SemaphoreType.DMA((2,))],
    )(_sc_gather_body)


def _sc_gather_body(x_hbm, is_hbm, id_hbm, xs_out, xd_out,
                    is_v, id_v, rs2, rd2, gs, gd):
    wid = lax.axis_index("s") * NC + lax.axis_index("c")
    pltpu.sync_copy(is_hbm.at[wid], is_v)
    pltpu.sync_copy(id_hbm.at[wid], id_v)
    base = wid * EPW
    pltpu.async_copy(x_hbm.at[is_v.at[0]], rs2.at[0], gs.at[0])
    pltpu.async_copy(x_hbm.at[id_v.at[0]], rd2.at[0], gd.at[0])

    def body(j, carry):
        b = lax.rem(j, 2)
        nb = 1 - b
        nj = jnp.minimum(j + 1, NCHUNK - 1)
        pltpu.async_copy(x_hbm.at[is_v.at[nj]], rs2.at[nb], gs.at[nb])
        pltpu.async_copy(x_hbm.at[id_v.at[nj]], rd2.at[nb], gd.at[nb])
        off = base + j * CH
        pltpu.make_async_copy(x_hbm.at[is_v.at[j]], rs2.at[b], gs.at[b]).wait()
        pltpu.sync_copy(rs2.at[b], xs_out.at[pl.ds(off, CH)])
        pltpu.make_async_copy(x_hbm.at[id_v.at[j]], rd2.at[b], gd.at[b]).wait()
        pltpu.sync_copy(rd2.at[b], xd_out.at[pl.ds(off, CH)])
        return carry

    lax.fori_loop(0, NCHUNK, body, 0)
    bl = NCHUNK % 2
    pltpu.make_async_copy(x_hbm.at[is_v.at[NCHUNK - 1]], rs2.at[bl],
                          gs.at[bl]).wait()
    pltpu.make_async_copy(x_hbm.at[id_v.at[NCHUNK - 1]], rd2.at[bl],
                          gd.at[bl]).wait()


EPT = E // NS
NCHUNK2 = EPT // CH
HALF = 5120
NPR = 5248
RPT = NPR // NS


@functools.cache
def _make_sc_scatter():
    mesh = plsc.VectorSubcoreMesh(core_axis_name="c", subcore_axis_name="s")
    return functools.partial(
        pl.kernel,
        out_type=jax.ShapeDtypeStruct((NC, NPR, D), jnp.float32),
        mesh=mesh,
        scratch_types=[pltpu.VMEM((NCHUNK2, CH), jnp.int32),
                       pltpu.VMEM((NCHUNK2, CH), jnp.int32),
                       pltpu.VMEM((2, CH, D), jnp.float32),
                       pltpu.SemaphoreType.DMA((2,)),
                       pltpu.VMEM_SHARED((NPR, D), jnp.float32)],
    )(_sc_scatter_body)


def _sc_scatter_body(se_hbm, de_hbm, is2, id2, z_hbm, out,
                     is_v, id_v, rv2, sems, acc):
    cid = lax.axis_index("c")
    tid = lax.axis_index("s")
    stripe = pl.ds(tid * RPT, RPT)
    pltpu.sync_copy(z_hbm.at[stripe], acc.at[stripe])
    pltpu.sync_copy(is2.at[cid, tid], is_v)
    pltpu.sync_copy(id2.at[cid, tid], id_v)
    plsc.subcore_barrier()
    base = tid * EPT
    bl = NCHUNK2 % 2

    def add_loop(val_hbm, idx_v):
        pltpu.async_copy(val_hbm.at[pl.ds(base, CH)], rv2.at[0], sems.at[0])

        def body(j, carry):
            b = lax.rem(j, 2)
            nb = 1 - b
            nj = jnp.minimum(j + 1, NCHUNK2 - 1)
            pltpu.async_copy(val_hbm.at[pl.ds(base + nj * CH, CH)],
                             rv2.at[nb], sems.at[nb])
            pltpu.make_async_copy(val_hbm.at[pl.ds(base, CH)],
                                  rv2.at[b], sems.at[b]).wait()
            pltpu.sync_copy(rv2.at[b], acc.at[idx_v.at[j]], add=True)
            return carry

        lax.fori_loop(0, NCHUNK2, body, 0)
        pltpu.make_async_copy(val_hbm.at[pl.ds(base, CH)],
                              rv2.at[bl], sems.at[bl]).wait()

    add_loop(se_hbm, is_v)
    add_loop(de_hbm, id_v)
    plsc.subcore_barrier()
    pltpu.sync_copy(acc.at[stripe], out.at[cid, stripe])


IR = E // D


def _tc_idx_body(i_ref, o_ref):
    i = i_ref[...]
    o_ref[0] = jnp.where(i < HALF, i, HALF)
    o_ref[1] = jnp.where(i >= HALF, i - HALF, HALF)


def _tc_idx(idx):
    return pl.pallas_call(
        _tc_idx_body,
        in_specs=[pl.BlockSpec((2, IR, D), lambda: (0, 0, 0))],
        out_specs=pl.BlockSpec((2, 2, IR, D), lambda: (0, 0, 0, 0)),
        out_shape=jax.ShapeDtypeStruct((2, 2, IR, D), jnp.int32),
    )(idx.reshape(2, IR, D))


def _tc_h_stats_body(xs_ref, e_ref, xd_ref, w1_ref, b1_ref,
                     h_ref, sums_ref, acc_ref):
    i = pl.program_id(0)
    h = (jnp.dot(xs_ref[...], w1_ref[0:D, :], preferred_element_type=jnp.float32)
         + jnp.dot(e_ref[...], w1_ref[D:D + DE, :], preferred_element_type=jnp.float32)
         + jnp.dot(xd_ref[...], w1_ref[D + DE:DIN, :], preferred_element_type=jnp.float32)
         + b1_ref[...])
    h_ref[...] = h

    @pl.when(i == 0)
    def _():
        acc_ref[...] = jnp.zeros_like(acc_ref)

    acc_ref[0:1, :] += jnp.sum(h, axis=0, keepdims=True)
    acc_ref[1:2, :] += jnp.sum(h * h, axis=0, keepdims=True)

    @pl.when(i == GRID_E - 1)
    def _():
        sums_ref[...] = acc_ref[...]


def _tc_h_stats(xs, e, xd, w1, b1):
    return pl.pallas_call(
        _tc_h_stats_body,
        grid=(GRID_E,),
        in_specs=[
            pl.BlockSpec((BE, D), lambda i: (i, 0)),
            pl.BlockSpec((BE, DE), lambda i: (i, 0)),
            pl.BlockSpec((BE, D), lambda i: (i, 0)),
            pl.BlockSpec((DIN, H), lambda i: (0, 0)),
            pl.BlockSpec((1, H), lambda i: (0, 0)),
        ],
        out_specs=[
            pl.BlockSpec((BE, H), lambda i: (i, 0)),
            pl.BlockSpec((8, H), lambda i: (0, 0)),
        ],
        out_shape=[jax.ShapeDtypeStruct((E, H), jnp.float32),
                   jax.ShapeDtypeStruct((8, H), jnp.float32)],
        scratch_shapes=[pltpu.VMEM((8, H), jnp.float32)],
    )(xs, e, xd, w1, b1)


def _tc_norm_body(h_ref, sums_ref, g_ref, bt_ref, w2_ref, b2_ref,
                  se_ref, ee_ref, de_ref):
    mean = sums_ref[0:1, :] * (1.0 / E)
    var = sums_ref[1:2, :] * (1.0 / E) - mean * mean
    alpha = g_ref[...] * lax.rsqrt(var + EPS)
    delta = bt_ref[...] - mean * alpha
    hn = jnp.maximum(h_ref[...] * alpha + delta, 0.0)
    o = jnp.dot(hn, w2_ref[...], preferred_element_type=jnp.float32) + b2_ref[...]
    se_ref[...] = o[:, 0:D]
    ee_ref[...] = o[:, D:2 * D]
    de_ref[...] = o[:, 2 * D:3 * D]


def _tc_norm(h, sums, g, bt, w2, b2):
    return pl.pallas_call(
        _tc_norm_body,
        grid=(GRID_E,),
        in_specs=[
            pl.BlockSpec((BE, H), lambda i: (i, 0)),
            pl.BlockSpec((8, H), lambda i: (0, 0)),
            pl.BlockSpec((1, H), lambda i: (0, 0)),
            pl.BlockSpec((1, H), lambda i: (0, 0)),
            pl.BlockSpec((H, DO), lambda i: (0, 0)),
            pl.BlockSpec((1, DO), lambda i: (0, 0)),
        ],
        out_specs=[
            pl.BlockSpec((BE, D), lambda i: (i, 0)),
            pl.BlockSpec((BE, D), lambda i: (i, 0)),
            pl.BlockSpec((BE, D), lambda i: (i, 0)),
        ],
        out_shape=[jax.ShapeDtypeStruct((E, D), jnp.float32),
                   jax.ShapeDtypeStruct((E, D), jnp.float32),
                   jax.ShapeDtypeStruct((E, D), jnp.float32)],
    )(h, sums, g, bt, w2, b2)


def _tc_final_body(p_ref, w3_ref, b3_ref, o_ref):
    o_ref[...] = (jnp.dot(p_ref[...], w3_ref[...],
                          preferred_element_type=jnp.float32)
                  + b3_ref[...])


def _tc_final(p, w3, b3):
    return pl.pallas_call(
        _tc_final_body,
        grid=(GRID_N,),
        in_specs=[
            pl.BlockSpec((BN, D), lambda i: (i, 0)),
            pl.BlockSpec((D, D), lambda i: (0, 0)),
            pl.BlockSpec((1, D), lambda i: (0, 0)),
        ],
        out_specs=pl.BlockSpec((BN, D), lambda i: (i, 0)),
        out_shape=jax.ShapeDtypeStruct((N, D), jnp.float32),
    )(p, w3, b3)


def kernel(node_embedding, edge_embedding, edge_index,
           W1, b1, bn_gamma, bn_beta, W2, b2, W3, b3):
    idx_s3 = edge_index[0].reshape(NW, NCHUNK, CH)
    idx_d3 = edge_index[1].reshape(NW, NCHUNK, CH)
    xs, xd = _make_sc_gather()(node_embedding, idx_s3, idx_d3)
    h, sums = _tc_h_stats(xs, edge_embedding, xd, W1, b1.reshape(1, H))
    src_e, edg_e, dst_e = _tc_norm(h, sums, bn_gamma.reshape(1, H),
                                   bn_beta.reshape(1, H), W2, b2.reshape(1, DO))
    idx2 = _tc_idx(edge_index)
    sh = (NC, NS, NCHUNK2, CH)
    is2 = idx2[:, 0].reshape(sh)
    id2 = idx2[:, 1].reshape(sh)
    zeros_n = jnp.zeros((NPR, D), jnp.float32)
    parts = _make_sc_scatter()(src_e, dst_e, is2, id2, zeros_n)
    p = jnp.concatenate([parts[0, :HALF], parts[1, :N - HALF]], axis=0)
    node_out = _tc_final(p, W3, b3.reshape(1, D))
    return node_out, edg_e
